# baseline (speedup 1.0000x reference)
import functools

import jax
import jax.numpy as jnp
from jax import lax
from jax.experimental import pallas as pl
from jax.experimental.pallas import tpu as pltpu
from jax.experimental.pallas import tpu_sc as plsc

_N = 4096
_NCOMBO = 4
_CHUNK = 512
_GROUPS = _CHUNK // 16
_GBLK = 8


def _newton_sqrt(x):
    i = lax.bitcast_convert_type(x, jnp.int32)
    y = lax.bitcast_convert_type(
        (i >> 1) + jnp.int32(0x1FBD3F7D), jnp.float32)
    half = jnp.float32(0.5)
    y = half * (y + x / y)
    y = half * (y + x / y)
    y = half * (y + x / y)
    return y


def _chamfer_sc(q_hbm, out_hbm, qx, qy, qz, cx, cy, cz, accv):
    nc = 2
    wid = lax.axis_index("s") * nc + lax.axis_index("c")
    combo = wid // 8
    chunk = wid % 8
    ccombo = combo ^ 1
    qbase = chunk * _CHUNK

    qrow = combo * (3 * _N)
    crow = ccombo * (3 * _N)
    pltpu.sync_copy(q_hbm.at[pl.ds(qrow, _N)], qx)
    pltpu.sync_copy(q_hbm.at[pl.ds(qrow + _N, _N)], qy)
    pltpu.sync_copy(q_hbm.at[pl.ds(qrow + 2 * _N, _N)], qz)
    pltpu.sync_copy(q_hbm.at[pl.ds(crow, _N)], cx)
    pltpu.sync_copy(q_hbm.at[pl.ds(crow + _N, _N)], cy)
    pltpu.sync_copy(q_hbm.at[pl.ds(crow + 2 * _N, _N)], cz)

    acc = jnp.zeros((16,), jnp.float32)
    big = jnp.full((16,), 3.0e38, jnp.float32)

    for blk in range(_GROUPS // _GBLK):
        qvs = []
        for g in range(_GBLK):
            off = qbase + (blk * _GBLK + g) * 16
            qvs.append((qx[pl.ds(off, 16)],
                        qy[pl.ds(off, 16)],
                        qz[pl.ds(off, 16)]))

        def body(jb, dmins, qvs=qvs):
            base = jb * 16
            cxv = cx[pl.ds(base, 16)]
            cyv = cy[pl.ds(base, 16)]
            czv = cz[pl.ds(base, 16)]
            out = list(dmins)
            for lane in range(16):
                bx = cxv[lane]
                by = cyv[lane]
                bz = czv[lane]
                for g in range(_GBLK):
                    dx = qvs[g][0] - bx
                    dy = qvs[g][1] - by
                    dz = qvs[g][2] - bz
                    d2 = dx * dx + dy * dy + dz * dz
                    out[g] = jnp.minimum(out[g], d2)
            return tuple(out)

        dmins = lax.fori_loop(0, _N // 16, body, tuple([big] * _GBLK))
        for g in range(_GBLK):
            acc = acc + _newton_sqrt(dmins[g])

    accv[...] = acc
    pltpu.sync_copy(accv, out_hbm.at[pl.ds(wid * 16, 16)])


def kernel(pc1, pc2):
    b = pc1.shape[0]
    n = pc1.shape[1]
    q = jnp.stack([pc1[0].T, pc2[0].T, pc1[1].T, pc2[1].T]).reshape(-1)

    mesh = plsc.VectorSubcoreMesh(core_axis_name="c", subcore_axis_name="s")
    run = functools.partial(
        pl.kernel,
        mesh=mesh,
        out_type=jax.ShapeDtypeStruct((32 * 16,), jnp.float32),
        scratch_types=[pltpu.VMEM((n,), jnp.float32)] * 6
        + [pltpu.VMEM((16,), jnp.float32)],
    )(_chamfer_sc)
    partials = run(q)
    return jnp.sum(partials) / jnp.float32(2 * n * b)
